# Initial kernel scaffold; baseline (speedup 1.0000x reference)
import jax
import jax.numpy as jnp
from jax.experimental import pallas as pl


def kernel(x, gate_W, expert_weights, expert_W, expert_b):
    raise NotImplementedError("write your pallas kernel here")


# speedup vs baseline: 3.1209x; 3.1209x over previous
import functools

import jax
import jax.numpy as jnp
from jax import lax
from jax.experimental import pallas as pl


TILE = 2048


def _pass1_body(x_ref, gw_ref, dvec_ref, ei_ref):
    i = pl.program_id(0)
    xt = x_ref[...]
    gw = gw_ref[...]
    E = gw.shape[0]

    logits = lax.dot_general(xt, gw, (((1,), (1,)), ((), ())),
                             preferred_element_type=jnp.float32)

    iota_e = lax.broadcasted_iota(jnp.int32, logits.shape, 1)
    m1 = jnp.max(logits, axis=1, keepdims=True)
    i1 = jnp.min(jnp.where(logits == m1, iota_e, E), axis=1, keepdims=True)
    mask1 = iota_e == i1
    neg = jnp.where(mask1, -jnp.inf, logits)
    m2 = jnp.max(neg, axis=1, keepdims=True)
    i2 = jnp.min(jnp.where(neg == m2, iota_e, E), axis=1, keepdims=True)
    mask2 = iota_e == i2

    e21 = jnp.exp(m2 - m1)
    w1 = 1.0 / (1.0 + e21)
    w2 = e21 * w1
    dvec = jnp.where(mask1, w1, jnp.where(mask2, w2, 0.0))
    dvec_ref[...] = dvec

    contrib = lax.dot_general(dvec, xt, (((0,), (0,)), ((), ())),
                              preferred_element_type=jnp.float32)

    @pl.when(i == 0)
    def _():
        ei_ref[...] = jnp.zeros_like(ei_ref)

    ei_ref[...] += contrib


def _expert_body(ei_ref, w_ref, bT_ref, yT_ref):
    E = ei_ref.shape[0]
    for e in range(E):
        w_e = w_ref[e]
        prod = w_e * ei_ref[e:e + 1, :]
        col = jnp.sum(prod, axis=1, keepdims=True)
        yT_ref[:, e:e + 1] = col + bT_ref[:, e:e + 1]


def _pass2_body(dvec_ref, yT_ref, out_ref):
    dvec = dvec_ref[...]
    yT = yT_ref[...]
    out_ref[...] = lax.dot_general(dvec, yT, (((1,), (1,)), ((), ())),
                                   preferred_element_type=jnp.float32)


@jax.jit
def _moe(x, gate_W, expert_weights, expert_W, expert_b):
    B, S, D = x.shape
    T = B * S
    E, F, _ = expert_W.shape
    x_flat = x.reshape(T, D)
    gw = gate_W * expert_weights[:, None]
    n_tiles = T // TILE

    dvec, ei = pl.pallas_call(
        _pass1_body,
        grid=(n_tiles,),
        in_specs=[
            pl.BlockSpec((TILE, D), lambda i: (i, 0)),
            pl.BlockSpec((E, D), lambda i: (0, 0)),
        ],
        out_specs=[
            pl.BlockSpec((TILE, E), lambda i: (i, 0)),
            pl.BlockSpec((E, D), lambda i: (0, 0)),
        ],
        out_shape=[
            jax.ShapeDtypeStruct((T, E), jnp.float32),
            jax.ShapeDtypeStruct((E, D), jnp.float32),
        ],
    )(x_flat, gw)

    yT = pl.pallas_call(
        _expert_body,
        grid=(1,),
        in_specs=[
            pl.BlockSpec((E, D), lambda i: (0, 0)),
            pl.BlockSpec((E, F, D), lambda i: (0, 0, 0)),
            pl.BlockSpec((F, E), lambda i: (0, 0)),
        ],
        out_specs=pl.BlockSpec((F, E), lambda i: (0, 0)),
        out_shape=jax.ShapeDtypeStruct((F, E), jnp.float32),
    )(ei, expert_W, expert_b.T)

    out = pl.pallas_call(
        _pass2_body,
        grid=(n_tiles,),
        in_specs=[
            pl.BlockSpec((TILE, E), lambda i: (i, 0)),
            pl.BlockSpec((F, E), lambda i: (0, 0)),
        ],
        out_specs=pl.BlockSpec((TILE, F), lambda i: (i, 0)),
        out_shape=jax.ShapeDtypeStruct((T, F), jnp.float32),
    )(dvec, yT)

    return out.reshape(B, S, F)


def kernel(x, gate_W, expert_weights, expert_W, expert_b):
    return _moe(x, gate_W, expert_weights, expert_W, expert_b)


# speedup vs baseline: 3.4684x; 1.1114x over previous
import functools

import jax
import jax.numpy as jnp
from jax import lax
from jax.experimental import pallas as pl


TILE = 2048


def _pass1_body(x_ref, gw_ref, dvec_ref, ei_ref):
    i = pl.program_id(0)
    xt = x_ref[...]
    gw = gw_ref[...]
    E = gw.shape[0]

    logits = lax.dot_general(xt, gw, (((1,), (1,)), ((), ())),
                             preferred_element_type=jnp.float32)

    m1 = jnp.max(logits, axis=1, keepdims=True)
    mask1 = logits == m1
    neg = jnp.where(mask1, -jnp.inf, logits)
    m2 = jnp.max(neg, axis=1, keepdims=True)
    mask2 = neg == m2

    e21 = jnp.exp(m2 - m1)
    w1 = 1.0 / (1.0 + e21)
    w2 = e21 * w1
    dvec = jnp.where(mask1, w1, jnp.where(mask2, w2, 0.0))
    dvec_ref[...] = dvec

    contrib = lax.dot_general(dvec, xt, (((0,), (0,)), ((), ())),
                              preferred_element_type=jnp.float32)

    @pl.when(i == 0)
    def _():
        ei_ref[...] = jnp.zeros_like(ei_ref)

    ei_ref[...] += contrib


def _expert_body(ei_ref, w_ref, bT_ref, yT_ref):
    E = ei_ref.shape[0]
    for e in range(E):
        w_e = w_ref[e]
        prod = w_e * ei_ref[e:e + 1, :]
        col = jnp.sum(prod, axis=1, keepdims=True)
        yT_ref[:, e:e + 1] = col + bT_ref[:, e:e + 1]


def _pass2_body(dvec_ref, yT_ref, out_ref):
    dvec = dvec_ref[...]
    yT = yT_ref[...]
    out_ref[...] = lax.dot_general(dvec, yT, (((1,), (1,)), ((), ())),
                                   preferred_element_type=jnp.float32)


@jax.jit
def _moe(x, gate_W, expert_weights, expert_W, expert_b):
    B, S, D = x.shape
    T = B * S
    E, F, _ = expert_W.shape
    x_flat = x.reshape(T, D)
    gw = gate_W * expert_weights[:, None]
    n_tiles = T // TILE

    dvec, ei = pl.pallas_call(
        _pass1_body,
        grid=(n_tiles,),
        in_specs=[
            pl.BlockSpec((TILE, D), lambda i: (i, 0)),
            pl.BlockSpec((E, D), lambda i: (0, 0)),
        ],
        out_specs=[
            pl.BlockSpec((TILE, E), lambda i: (i, 0)),
            pl.BlockSpec((E, D), lambda i: (0, 0)),
        ],
        out_shape=[
            jax.ShapeDtypeStruct((T, E), jnp.float32),
            jax.ShapeDtypeStruct((E, D), jnp.float32),
        ],
    )(x_flat, gw)

    yT = pl.pallas_call(
        _expert_body,
        grid=(1,),
        in_specs=[
            pl.BlockSpec((E, D), lambda i: (0, 0)),
            pl.BlockSpec((E, F, D), lambda i: (0, 0, 0)),
            pl.BlockSpec((F, E), lambda i: (0, 0)),
        ],
        out_specs=pl.BlockSpec((F, E), lambda i: (0, 0)),
        out_shape=jax.ShapeDtypeStruct((F, E), jnp.float32),
    )(ei, expert_W, expert_b.T)

    out = pl.pallas_call(
        _pass2_body,
        grid=(n_tiles,),
        in_specs=[
            pl.BlockSpec((TILE, E), lambda i: (i, 0)),
            pl.BlockSpec((F, E), lambda i: (0, 0)),
        ],
        out_specs=pl.BlockSpec((TILE, F), lambda i: (i, 0)),
        out_shape=jax.ShapeDtypeStruct((T, F), jnp.float32),
    )(dvec, yT)

    return out.reshape(B, S, F)


def kernel(x, gate_W, expert_weights, expert_W, expert_b):
    return _moe(x, gate_W, expert_weights, expert_W, expert_b)


# speedup vs baseline: 3.6840x; 1.0622x over previous
import functools

import jax
import jax.numpy as jnp
from jax import lax
from jax.experimental import pallas as pl


TILE = 4096


def _pass1_body(x_ref, gw_ref, dvec_ref, ei_ref):
    i = pl.program_id(0)
    xt = x_ref[...]
    gw = gw_ref[...]
    E = gw.shape[0]

    logits = lax.dot_general(xt, gw, (((1,), (1,)), ((), ())),
                             preferred_element_type=jnp.float32)

    m1 = jnp.max(logits, axis=1, keepdims=True)
    mask1 = logits == m1
    neg = jnp.where(mask1, -jnp.inf, logits)
    m2 = jnp.max(neg, axis=1, keepdims=True)
    mask2 = neg == m2

    e21 = jnp.exp(m2 - m1)
    w1 = 1.0 / (1.0 + e21)
    w2 = e21 * w1
    dvec = jnp.where(mask1, w1, jnp.where(mask2, w2, 0.0))
    dvec_ref[...] = dvec

    contrib = lax.dot_general(dvec, xt, (((0,), (0,)), ((), ())),
                              preferred_element_type=jnp.float32)

    @pl.when(i == 0)
    def _():
        ei_ref[...] = jnp.zeros_like(ei_ref)

    ei_ref[...] += contrib


def _expert_body(ei_ref, w_ref, bT_ref, yT_ref):
    E = ei_ref.shape[0]
    for e in range(E):
        w_e = w_ref[e]
        prod = w_e * ei_ref[e:e + 1, :]
        col = jnp.sum(prod, axis=1, keepdims=True)
        yT_ref[:, e:e + 1] = col + bT_ref[:, e:e + 1]


def _pass2_body(dvec_ref, yT_ref, out_ref):
    dvec = dvec_ref[...]
    yT = yT_ref[...]
    out_ref[...] = lax.dot_general(dvec, yT, (((1,), (1,)), ((), ())),
                                   preferred_element_type=jnp.float32)


@jax.jit
def _moe(x, gate_W, expert_weights, expert_W, expert_b):
    B, S, D = x.shape
    T = B * S
    E, F, _ = expert_W.shape
    x_flat = x.reshape(T, D)
    gw = gate_W * expert_weights[:, None]
    n_tiles = T // TILE

    dvec, ei = pl.pallas_call(
        _pass1_body,
        grid=(n_tiles,),
        in_specs=[
            pl.BlockSpec((TILE, D), lambda i: (i, 0)),
            pl.BlockSpec((E, D), lambda i: (0, 0)),
        ],
        out_specs=[
            pl.BlockSpec((TILE, E), lambda i: (i, 0)),
            pl.BlockSpec((E, D), lambda i: (0, 0)),
        ],
        out_shape=[
            jax.ShapeDtypeStruct((T, E), jnp.float32),
            jax.ShapeDtypeStruct((E, D), jnp.float32),
        ],
    )(x_flat, gw)

    yT = pl.pallas_call(
        _expert_body,
        grid=(1,),
        in_specs=[
            pl.BlockSpec((E, D), lambda i: (0, 0)),
            pl.BlockSpec((E, F, D), lambda i: (0, 0, 0)),
            pl.BlockSpec((F, E), lambda i: (0, 0)),
        ],
        out_specs=pl.BlockSpec((F, E), lambda i: (0, 0)),
        out_shape=jax.ShapeDtypeStruct((F, E), jnp.float32),
    )(ei, expert_W, expert_b.T)

    out = pl.pallas_call(
        _pass2_body,
        grid=(n_tiles,),
        in_specs=[
            pl.BlockSpec((TILE, E), lambda i: (i, 0)),
            pl.BlockSpec((F, E), lambda i: (0, 0)),
        ],
        out_specs=pl.BlockSpec((TILE, F), lambda i: (i, 0)),
        out_shape=jax.ShapeDtypeStruct((T, F), jnp.float32),
    )(dvec, yT)

    return out.reshape(B, S, F)


def kernel(x, gate_W, expert_weights, expert_W, expert_b):
    return _moe(x, gate_W, expert_weights, expert_W, expert_b)


# speedup vs baseline: 4.4842x; 1.2172x over previous
import jax
import jax.numpy as jnp
from jax import lax
from jax.experimental import pallas as pl
from jax.experimental.pallas import tpu as pltpu


TILE = 2048


def _body(x_ref, gw_ref, w_ref, b_ref, out_ref, dvec_s, ei_s, y_s):
    p = pl.program_id(0)
    i = pl.program_id(1)
    E = gw_ref.shape[0]

    @pl.when(p == 0)
    def _phase0():
        xt = x_ref[...]
        gw = gw_ref[...]
        logits = lax.dot_general(gw, xt, (((1,), (1,)), ((), ())),
                                 preferred_element_type=jnp.float32)
        m1 = jnp.max(logits, axis=0, keepdims=True)
        mask1 = logits == m1
        neg = jnp.where(mask1, -jnp.inf, logits)
        m2 = jnp.max(neg, axis=0, keepdims=True)
        mask2 = neg == m2
        e21 = jnp.exp(m2 - m1)
        w1 = 1.0 / (1.0 + e21)
        w2 = e21 * w1
        dvec = jnp.where(mask1, w1, jnp.where(mask2, w2, 0.0))
        dvec_s[:, pl.ds(i * TILE, TILE)] = dvec
        contrib = lax.dot_general(dvec, xt, (((1,), (0,)), ((), ())),
                                  preferred_element_type=jnp.float32)

        @pl.when(i == 0)
        def _():
            ei_s[...] = jnp.zeros_like(ei_s)

        ei_s[...] += contrib

    @pl.when((p == 1) & (i == 0))
    def _expert():
        for e in range(E):
            row = lax.dot_general(ei_s[e:e + 1, :], w_ref[e],
                                  (((1,), (1,)), ((), ())),
                                  preferred_element_type=jnp.float32)
            y_s[e:e + 1, :] = row + b_ref[e:e + 1, :]

    @pl.when(p == 1)
    def _phase1():
        dvec = dvec_s[:, pl.ds(i * TILE, TILE)]
        out_ref[...] = lax.dot_general(dvec, y_s[...], (((0,), (0,)), ((), ())),
                                       preferred_element_type=jnp.float32)


@jax.jit
def _moe(x, gate_W, expert_weights, expert_W, expert_b):
    B, S, D = x.shape
    T = B * S
    E, F, _ = expert_W.shape
    x_flat = x.reshape(T, D)
    gw = gate_W * expert_weights[:, None]
    n_tiles = T // TILE

    out = pl.pallas_call(
        _body,
        grid=(2, n_tiles),
        in_specs=[
            pl.BlockSpec((TILE, D), lambda p, i: ((1 - p) * i + p * (T // TILE - 1), 0)),
            pl.BlockSpec((E, D), lambda p, i: (0, 0)),
            pl.BlockSpec((E, F, D), lambda p, i: (0, 0, 0)),
            pl.BlockSpec((E, F), lambda p, i: (0, 0)),
        ],
        out_specs=pl.BlockSpec((TILE, F), lambda p, i: (p * i, 0)),
        out_shape=jax.ShapeDtypeStruct((T, F), jnp.float32),
        scratch_shapes=[
            pltpu.VMEM((E, T), jnp.float32),
            pltpu.VMEM((E, D), jnp.float32),
            pltpu.VMEM((E, F), jnp.float32),
        ],
    )(x_flat, gw, expert_W, expert_b)

    return out.reshape(B, S, F)


def kernel(x, gate_W, expert_weights, expert_W, expert_b):
    return _moe(x, gate_W, expert_weights, expert_W, expert_b)
